# Initial kernel scaffold; baseline (speedup 1.0000x reference)
import jax
import jax.numpy as jnp
from jax.experimental import pallas as pl


def kernel(local_x, local_edge_index, node_cluster, node_ratio, voxel_x, voxel_edge_index, voxel_level, cross_edge_index, program_noise, voxel_noise, params):
    raise NotImplementedError("write your pallas kernel here")


# speedup vs baseline: 1.0465x; 1.0465x over previous
import functools

import jax
import jax.numpy as jnp
import numpy as np
from jax import lax
from jax.experimental import pallas as pl
from jax.experimental.pallas import tpu as pltpu

H = 128
NPN = 10000
NVX = 10000
NCL = 500
P_STEPS = 3
V_STEPS = 4
EPE = 160000
EVE = 320000
ECE = 320000


def _fused_body(nx, npost, has_b, act, out_is_col, *refs):
    xrefs = refs[:nx]
    wrefs = refs[nx:2 * nx]
    i = 2 * nx
    bref = None
    if has_b:
        bref = refs[i]
        i += 1
    prefs = refs[i:i + npost]
    oref = refs[-1]
    acc = None
    for xr, wr in zip(xrefs, wrefs):
        part = jnp.dot(xr[...], wr[...], preferred_element_type=jnp.float32,
                       precision=lax.Precision.HIGHEST)
        acc = part if acc is None else acc + part
    if bref is not None:
        acc = acc + bref[...]
    if act == "lrelu":
        acc = jnp.maximum(acc, 0.01 * acc)
    elif act == "sigmoid":
        acc = jax.nn.sigmoid(acc)
    for prr in prefs:
        acc = acc + prr[...]
    oref[...] = acc


def _fused(xs, ws, b=None, post=(), act=None, bm=2000):
    n = xs[0].shape[0]
    out_w = ws[0].shape[1]
    grid = n // bm
    assert grid * bm == n
    in_specs = []
    args = []
    for x in xs:
        k = x.shape[1]
        in_specs.append(pl.BlockSpec((bm, k), lambda i: (i, 0)))
        args.append(x)
    for w in ws:
        in_specs.append(pl.BlockSpec(w.shape, lambda i: (0, 0)))
        args.append(w)
    if b is not None:
        b2 = b.reshape(1, out_w)
        in_specs.append(pl.BlockSpec((1, out_w), lambda i: (0, 0)))
        args.append(b2)
    for p in post:
        in_specs.append(pl.BlockSpec((bm, p.shape[1]), lambda i: (i, 0)))
        args.append(p)
    body = functools.partial(_fused_body, len(xs), len(post), b is not None,
                             act, out_w == 1)
    return pl.pallas_call(
        body,
        grid=(grid,),
        in_specs=in_specs,
        out_specs=pl.BlockSpec((bm, out_w), lambda i: (i, 0)),
        out_shape=jax.ShapeDtypeStruct((n, out_w), jnp.float32),
    )(*args)


def _edge_e_body(z_ref, th_ref, o_ref):
    z = z_ref[...]
    t = jnp.tanh(z)
    o_ref[...] = jnp.dot(t, th_ref[...], preferred_element_type=jnp.float32,
                         precision=lax.Precision.HIGHEST)


def _edge_e(z, theta, bm=6400):
    e = z.shape[0]
    grid = e // bm
    assert grid * bm == e
    return pl.pallas_call(
        _edge_e_body,
        grid=(grid,),
        in_specs=[pl.BlockSpec((bm, H), lambda i: (i, 0)),
                  pl.BlockSpec((H, 1), lambda i: (0, 0))],
        out_specs=pl.BlockSpec((bm, 1), lambda i: (i, 0)),
        out_shape=jax.ShapeDtypeStruct((e, 1), jnp.float32),
    )(z, theta)


def _pe_tab():
    pos = np.arange(100, dtype=np.float32)[:, None]
    i2 = np.arange(0, H, 2, dtype=np.float32)
    ang = pos / (10000.0 ** (i2 / H))
    t = np.zeros((100, H), dtype=np.float32)
    t[:, 0::2] = np.sin(ang)
    t[:, 1::2] = np.cos(ang)
    return jnp.asarray(t)


def _seg_sum(x, idx, n):
    return jax.ops.segment_sum(x, idx, num_segments=n)


def _inv_counts(idx, n):
    cnt = jax.ops.segment_sum(jnp.ones(idx.shape, jnp.float32), idx,
                              num_segments=n)
    return 1.0 / jnp.maximum(cnt, 1.0)


def kernel(local_x, local_edge_index, node_cluster, node_ratio, voxel_x,
           voxel_edge_index, voxel_level, cross_edge_index, program_noise,
           voxel_noise, params):
    p = params
    src = local_edge_index[0]
    dst = local_edge_index[1]
    ratio = jnp.sum(node_ratio, axis=1)[:, None]
    pinv = _inv_counts(dst, NPN)[:, None]
    cinv = _inv_counts(node_cluster, NCL)[:, None]

    Wp = p["p_enc"]["W"]
    x = _fused([local_x, program_noise], [Wp[:128], Wp[128:]],
               p["p_enc"]["b"], act="lrelu")

    for l in range(P_STEPS):
        Wm = p["p_msg"][l]["W"]
        F = _fused([x], [Wm[:128]], p["p_msg"][l]["b"])
        G = _fused([x], [Wm[128:]])
        m = jnp.maximum(F[dst] + G[src], 0.01 * (F[dst] + G[src]))
        aggr = _seg_sum(m, dst, NPN) * pinv
        cm = _seg_sum(x, node_cluster, NCL) * cinv
        c = cm[node_cluster] * ratio
        Wu = p["p_upd"][l]["W"]
        x = _fused([x, aggr, c], [Wu[:128], Wu[128:256], Wu[256:]],
                   p["p_upd"][l]["b"], post=[x], act="lrelu")

    pe = _pe_tab()
    pos = pe[voxel_level]
    Wv = p["v_enc"]["W"]
    v = _fused([voxel_x, voxel_noise], [Wv[:128], Wv[128:]],
               p["v_enc"]["b"], post=[pos], act="lrelu")
    vsrc = voxel_edge_index[0]
    vdst = voxel_edge_index[1]

    ce0 = cross_edge_index[0]
    ce1 = cross_edge_index[1]
    ptr = p["ptr"]
    theta = ptr["theta"]

    for li in range(V_STEPS):
        Wm = p["v_msg"][li]["W"]
        D = _fused([v, pos], [Wm[:128], Wm[256:]], p["v_msg"][li]["b"])
        S = _fused([v, pos], [Wm[128:256], -Wm[256:]])
        t = D[vdst] + S[vsrc]
        msg = jnp.maximum(t, 0.01 * t)
        aggr = _seg_sum(msg, vdst, NVX)
        Wu = p["v_upd"][li]["W"]
        v = _fused([v, aggr], [Wu[:128], Wu[128:]], p["v_upd"][li]["b"],
                   post=[v], act="lrelu")
        if (li + 1) % 2 == 0:
            h = _fused([v], [ptr["m1"]["W"]], ptr["m1"]["b"], act="lrelu")
            mask = _fused([h], [ptr["m2"]["W"]], ptr["m2"]["b"],
                          act="sigmoid")
            Ptab = _fused([x], [ptr["Wp"]["W"]],
                          ptr["Wp"]["b"] + ptr["Wv"]["b"])
            Qtab = _fused([v], [ptr["Wv"]["W"]])
            z = Ptab[ce0] + Qtab[ce1]
            e = _edge_e(z, theta)[:, 0]
            u = jax.random.uniform(
                jax.random.fold_in(jax.random.key(42), li), (ECE,),
                minval=1e-9, maxval=1.0, dtype=jnp.float32)
            g = -jnp.log(-jnp.log(u))
            y = jax.nn.softmax(e + g, axis=0)
            summed = _seg_sum(x[ce0] * y[:, None], ce1, NVX)
            v = v + mask * summed
    return v


# speedup vs baseline: 1.5308x; 1.4627x over previous
import functools

import jax
import jax.numpy as jnp
import numpy as np
from jax import lax
from jax.experimental import pallas as pl
from jax.experimental.pallas import tpu as pltpu
from jax.experimental.pallas import tpu_sc as plsc

NW = 32

H = 128
NPN = 10000
NVX = 10000
NCL = 500
P_STEPS = 3
V_STEPS = 4
EPE = 160000
EVE = 320000
ECE = 320000


def _fused_body(nx, npost, has_b, act, refs):
    xrefs = refs[:nx]
    wrefs = refs[nx:2 * nx]
    i = 2 * nx
    bref = None
    if has_b:
        bref = refs[i]
        i += 1
    prefs = refs[i:i + npost]
    oref = refs[-1]
    acc = None
    for xr, wr in zip(xrefs, wrefs):
        part = jnp.dot(xr[...], wr[...], preferred_element_type=jnp.float32,
                       precision=lax.Precision.HIGHEST)
        acc = part if acc is None else acc + part
    if bref is not None:
        acc = acc + bref[...]
    if act == "lrelu":
        acc = jnp.maximum(acc, 0.01 * acc)
    elif act == "sigmoid":
        acc = jax.nn.sigmoid(acc)
    for prr in prefs:
        acc = acc + prr[...]
    oref[...] = acc


def _fused(xs, ws, b=None, post=(), act=None, bm=2000):
    n = xs[0].shape[0]
    out_w = ws[0].shape[1]
    grid = n // bm
    assert grid * bm == n
    in_specs = []
    args = []
    for x in xs:
        k = x.shape[1]
        in_specs.append(pl.BlockSpec((bm, k), lambda i: (i, 0)))
        args.append(x)
    for w in ws:
        in_specs.append(pl.BlockSpec(w.shape, lambda i: (0, 0)))
        args.append(w)
    if b is not None:
        b2 = b.reshape(1, out_w)
        in_specs.append(pl.BlockSpec((1, out_w), lambda i: (0, 0)))
        args.append(b2)
    for po in post:
        in_specs.append(pl.BlockSpec((bm, po.shape[1]), lambda i: (i, 0)))
        args.append(po)

    def body(*refs):
        _fused_body(len(xs), len(post), b is not None, act, refs)

    return pl.pallas_call(
        body,
        grid=(grid,),
        in_specs=in_specs,
        out_specs=pl.BlockSpec((bm, out_w), lambda i: (i, 0)),
        out_shape=jax.ShapeDtypeStruct((n, out_w), jnp.float32),
    )(*args)


def _edge_e_body(zp_ref, zq_ref, th_ref, o_ref):
    t = jnp.tanh(zp_ref[...] + zq_ref[...])
    o_ref[...] = jnp.dot(t, th_ref[...], preferred_element_type=jnp.float32,
                         precision=lax.Precision.HIGHEST)


def _edge_e(zp, zq, theta, bm=6400):
    e = zp.shape[0]
    grid = e // bm
    assert grid * bm == e
    return pl.pallas_call(
        _edge_e_body,
        grid=(grid,),
        in_specs=[pl.BlockSpec((bm, H), lambda i: (i, 0)),
                  pl.BlockSpec((bm, H), lambda i: (i, 0)),
                  pl.BlockSpec((H, 1), lambda i: (0, 0))],
        out_specs=pl.BlockSpec((bm, 1), lambda i: (i, 0)),
        out_shape=jax.ShapeDtypeStruct((e, 1), jnp.float32),
    )(zp, zq, theta)


def _sc_mesh():
    return plsc.VectorSubcoreMesh(core_axis_name="c", subcore_axis_name="s")


def _mk_sc_gather(nch, ch):
    @functools.partial(
        pl.kernel, mesh=_sc_mesh(),
        out_type=jax.ShapeDtypeStruct((NW * nch, ch, H), jnp.float32),
        scratch_types=[
            pltpu.VMEM((nch, ch), jnp.int32),
            pltpu.VMEM((ch, H), jnp.float32),
            pltpu.SemaphoreType.DMA,
        ],
    )
    def k(table, idx3, out, idx_v, rows_v, sem):
        wid = lax.axis_index("s") * 2 + lax.axis_index("c")
        pltpu.sync_copy(idx3.at[wid], idx_v)

        def body(t, _):
            pltpu.async_copy(table.at[idx_v.at[t]], rows_v, sem).wait()
            pltpu.sync_copy(rows_v, out.at[wid * nch + t])
            return 0

        lax.fori_loop(0, nch, body, 0, unroll=False)

    return k


def _mk_sc_msg(nch, ch):
    @functools.partial(
        pl.kernel, mesh=_sc_mesh(),
        out_type=jax.ShapeDtypeStruct((NW * nch, ch, H), jnp.float32),
        scratch_types=[
            pltpu.VMEM((nch, ch), jnp.int32),
            pltpu.VMEM((nch, ch), jnp.int32),
            pltpu.VMEM((ch, H), jnp.float32),
            pltpu.VMEM((ch, H), jnp.float32),
            pltpu.SemaphoreType.DMA,
            pltpu.SemaphoreType.DMA,
        ],
    )
    def k(tabd, tabs, idxd3, idxs3, out, idxd_v, idxs_v, rowsd, rowss,
          semd, sems):
        wid = lax.axis_index("s") * 2 + lax.axis_index("c")
        pltpu.sync_copy(idxd3.at[wid], idxd_v)
        pltpu.sync_copy(idxs3.at[wid], idxs_v)

        def body(t, _):
            cp1 = pltpu.async_copy(tabd.at[idxd_v.at[t]], rowsd, semd)
            cp2 = pltpu.async_copy(tabs.at[idxs_v.at[t]], rowss, sems)
            cp1.wait()
            cp2.wait()

            def edge(r, _):
                for j in range(H // 16):
                    sl = pl.ds(j * 16, 16)
                    val = rowsd[r, sl] + rowss[r, sl]
                    rowsd[r, sl] = jnp.maximum(val, 0.01 * val)
                return 0

            lax.fori_loop(0, ch, edge, 0, unroll=False)
            pltpu.sync_copy(rowsd, out.at[wid * nch + t])
            return 0

        lax.fori_loop(0, nch, body, 0, unroll=False)

    return k


def _mk_sc_gather_scale(nch, ch):
    @functools.partial(
        pl.kernel, mesh=_sc_mesh(),
        out_type=jax.ShapeDtypeStruct((NW * nch, ch, H), jnp.float32),
        scratch_types=[
            pltpu.VMEM((nch, ch), jnp.int32),
            pltpu.VMEM((ch, 16), jnp.float32),
            pltpu.VMEM((ch, H), jnp.float32),
            pltpu.SemaphoreType.DMA,
        ],
    )
    def k(tabg, idxg3, y4, out, idxg_v, y_v, rows, sem):
        wid = lax.axis_index("s") * 2 + lax.axis_index("c")
        pltpu.sync_copy(idxg3.at[wid], idxg_v)

        def body(t, _):
            cp1 = pltpu.async_copy(tabg.at[idxg_v.at[t]], rows, sem)
            pltpu.sync_copy(y4.at[wid, t], y_v)
            cp1.wait()

            def edge(r, _):
                yv = y_v[r, :]
                for j in range(H // 16):
                    sl = pl.ds(j * 16, 16)
                    rows[r, sl] = rows[r, sl] * yv
                return 0

            lax.fori_loop(0, ch, edge, 0, unroll=False)
            pltpu.sync_copy(rows, out.at[wid * nch + t])
            return 0

        lax.fori_loop(0, nch, body, 0, unroll=False)

    return k


def _rw(idx, nch, ch):
    return idx.reshape(NW, nch, ch)


def _r4(y, nch, ch):
    yb = jnp.broadcast_to(y[:, None], (y.shape[0], 16))
    return yb.reshape(NW, nch, ch, 16)


def _pe_tab():
    pos = np.arange(100, dtype=np.float32)[:, None]
    i2 = np.arange(0, H, 2, dtype=np.float32)
    ang = pos / (10000.0 ** (i2 / H))
    t = np.zeros((100, H), dtype=np.float32)
    t[:, 0::2] = np.sin(ang)
    t[:, 1::2] = np.cos(ang)
    return jnp.asarray(t)


def _seg_sum(x, idx, n):
    return jax.ops.segment_sum(x, idx, num_segments=n)


def _inv_counts(idx, n):
    cnt = jax.ops.segment_sum(jnp.ones(idx.shape, jnp.float32), idx,
                              num_segments=n)
    return (1.0 / jnp.maximum(cnt, 1.0))[:, None]


def kernel(local_x, local_edge_index, node_cluster, node_ratio, voxel_x,
           voxel_edge_index, voxel_level, cross_edge_index, program_noise,
           voxel_noise, params):
    p = params
    src = local_edge_index[0]
    dst = local_edge_index[1]
    ratio = jnp.sum(node_ratio, axis=1)[:, None]
    pinv = _inv_counts(dst, NPN)
    cinv = _inv_counts(node_cluster, NCL)

    msg_p = _mk_sc_msg(50, 100)
    msg_v = _mk_sc_msg(100, 100)
    gat_ptr = _mk_sc_gather(100, 100)
    gscale_ptr = _mk_sc_gather_scale(100, 100)
    src3 = _rw(src, 50, 100)
    dst3 = _rw(dst, 50, 100)
    vsrc3 = _rw(voxel_edge_index[0], 100, 100)
    vdst3 = _rw(voxel_edge_index[1], 100, 100)
    ce0 = cross_edge_index[0]
    ce1 = cross_edge_index[1]
    ce0_3 = _rw(ce0, 100, 100)
    ce1_3 = _rw(ce1, 100, 100)

    Wp = p["p_enc"]["W"]
    x = _fused([local_x, program_noise], [Wp[:128], Wp[128:]],
               p["p_enc"]["b"], act="lrelu")

    for l in range(P_STEPS):
        Wm = p["p_msg"][l]["W"]
        F = _fused([x], [Wm[:128]], p["p_msg"][l]["b"])
        G = _fused([x], [Wm[128:]])
        m = msg_p(F, G, dst3, src3).reshape(EPE, H)
        aggr = _seg_sum(m, dst, NPN) * pinv
        cm = _seg_sum(x, node_cluster, NCL) * cinv
        c = cm[node_cluster] * ratio
        Wu = p["p_upd"][l]["W"]
        x = _fused([x, aggr, c], [Wu[:128], Wu[128:256], Wu[256:]],
                   p["p_upd"][l]["b"], post=[x], act="lrelu")

    pe = _pe_tab()
    pos = pe[voxel_level]
    Wv = p["v_enc"]["W"]
    v = _fused([voxel_x, voxel_noise], [Wv[:128], Wv[128:]],
               p["v_enc"]["b"], post=[pos], act="lrelu")
    ptr = p["ptr"]
    theta = ptr["theta"]

    for li in range(V_STEPS):
        Wm = p["v_msg"][li]["W"]
        D = _fused([v, pos], [Wm[:128], Wm[256:]], p["v_msg"][li]["b"])
        S = _fused([v, pos], [Wm[128:256], -Wm[256:]])
        msg = msg_v(D, S, vdst3, vsrc3).reshape(EVE, H)
        aggr = _seg_sum(msg, voxel_edge_index[1], NVX)
        Wu = p["v_upd"][li]["W"]
        v = _fused([v, aggr], [Wu[:128], Wu[128:]], p["v_upd"][li]["b"],
                   post=[v], act="lrelu")
        if (li + 1) % 2 == 0:
            h = _fused([v], [ptr["m1"]["W"]], ptr["m1"]["b"], act="lrelu")
            mask = _fused([h], [ptr["m2"]["W"]], ptr["m2"]["b"],
                          act="sigmoid")
            Ptab = _fused([x], [ptr["Wp"]["W"]],
                          ptr["Wp"]["b"] + ptr["Wv"]["b"])
            Qtab = _fused([v], [ptr["Wv"]["W"]])
            zp = gat_ptr(Ptab, ce0_3).reshape(ECE, H)
            zq = gat_ptr(Qtab, ce1_3).reshape(ECE, H)
            e = _edge_e(zp, zq, theta)[:, 0]
            u = jax.random.uniform(
                jax.random.fold_in(jax.random.key(42), li), (ECE,),
                minval=1e-9, maxval=1.0, dtype=jnp.float32)
            g = -jnp.log(-jnp.log(u))
            y = jax.nn.softmax(e + g, axis=0)
            xy = gscale_ptr(x, ce0_3, _r4(y, 100, 100)).reshape(ECE, H)
            v = v + mask * _seg_sum(xy, ce1, NVX)
    return v
